# Initial kernel scaffold; baseline (speedup 1.0000x reference)
import jax
import jax.numpy as jnp
from jax.experimental import pallas as pl


def kernel(node_features, edge_features, senders, receivers, W1e, b1e, W2e, b2e, ge, be, W1n, b1n, W2n, b2n, gn, bn):
    raise NotImplementedError("write your pallas kernel here")


# speedup vs baseline: 2.8529x; 2.8529x over previous
import functools

import jax
import jax.numpy as jnp
from jax import lax
from jax.experimental import pallas as pl
from jax.experimental.pallas import tpu as pltpu
from jax.experimental.pallas import tpu_sc as plsc

_N = 10000
_E = 160000
_D = 256
_H = 128
_GW = 128
_SG = _E // _GW
_NSUB = 16
_NP = 10240
_RS = _NP // _NSUB
_EB = 1000
_NB = 1000
_EPS = 1e-5

_f32 = jnp.float32
_bf16 = jnp.bfloat16


def _layer_norm(y, g, b):
    mu = jnp.mean(y, axis=-1, keepdims=True)
    d = y - mu
    var = jnp.mean(d * d, axis=-1, keepdims=True)
    return d * lax.rsqrt(var + _EPS) * g + b


def _proj_body(x_ref, ws_ref, wr_ref, wn_ref, ps_ref, pr_ref, na_ref):
    x = x_ref[...].astype(_bf16)
    ps_ref[...] = jnp.dot(x, ws_ref[...], preferred_element_type=_f32)
    pr_ref[...] = jnp.dot(x, wr_ref[...], preferred_element_type=_f32)
    na_ref[...] = jnp.dot(x, wn_ref[...], preferred_element_type=_f32)


def _proj(nodes, ws, wr, wn):
    full = lambda shp: pl.BlockSpec(shp, lambda i: (0, 0))
    o = jax.ShapeDtypeStruct((_N, _D), _f32)
    return pl.pallas_call(
        _proj_body,
        grid=(_N // _NB,),
        in_specs=[pl.BlockSpec((_NB, _D), lambda i: (i, 0)),
                  full((_D, _D)), full((_D, _D)), full((_D, _D))],
        out_specs=[pl.BlockSpec((_NB, _D), lambda i: (i, 0))] * 3,
        out_shape=[o, o, o],
    )(nodes, ws, wr, wn)


def _sc_gather(table, idx2d):
    mesh = plsc.VectorSubcoreMesh(core_axis_name="c", subcore_axis_name="s")

    @functools.partial(
        pl.kernel,
        out_type=jax.ShapeDtypeStruct((_E, _D), _f32),
        mesh=mesh)
    def gk(t_hbm, i_hbm, o_hbm):
        def body(i_vmem, o_vmem):
            pltpu.sync_copy(t_hbm.at[i_vmem.at[0]], o_vmem)

        pltpu.emit_pipeline(
            body,
            grid=(_E // _GW,),
            in_specs=[pl.BlockSpec((1, _GW), lambda i: (0, i))],
            out_specs=[pl.BlockSpec((_GW, _D), lambda i: (i, 0))],
            core_axis_name=("c", "s"),
            dimension_semantics=(pltpu.PARALLEL,),
        )(i_hbm, o_hbm)

    return gk(table, idx2d)


def _edge_body(gs_ref, gr_ref, e_ref, w1_ref, w2_ref, b1_ref, b2_ref,
               g_ref, b_ref, oe_ref, lo_ref, hi_ref):
    e = e_ref[...]
    x = gs_ref[...] + gr_ref[...] + b1_ref[...]
    x = x + jnp.dot(e.astype(_bf16), w1_ref[...], preferred_element_type=_f32)
    h = jnp.maximum(x, 0.0)
    y = jnp.dot(h.astype(_bf16), w2_ref[...], preferred_element_type=_f32)
    ne = _layer_norm(y + b2_ref[...], g_ref[...], b_ref[...])
    oe_ref[...] = ne + e
    lo_ref[...] = ne[:, :_H]
    hi_ref[...] = ne[:, _H:]


def _edge_mlp(gs, gr, e, w1c, w2e, b1, b2, g, b):
    full = lambda shp: pl.BlockSpec(shp, lambda i: (0, 0))
    return pl.pallas_call(
        _edge_body,
        grid=(_E // _EB,),
        in_specs=[pl.BlockSpec((_EB, _D), lambda i: (i, 0))] * 3 +
                 [full((_D, _D)), full((_D, _D)),
                  full((1, _D)), full((1, _D)), full((1, _D)), full((1, _D))],
        out_specs=[pl.BlockSpec((_EB, _D), lambda i: (i, 0)),
                   pl.BlockSpec((_EB, _H), lambda i: (i, 0)),
                   pl.BlockSpec((_EB, _H), lambda i: (i, 0))],
        out_shape=[jax.ShapeDtypeStruct((_E, _D), _f32),
                   jax.ShapeDtypeStruct((_E, _H), _f32),
                   jax.ShapeDtypeStruct((_E, _H), _f32)],
    )(gs, gr, e, w1c, w2e, b1, b2, g, b)


def _sc_scatter(ne_lo, ne_hi, recv2d, zeros):
    mesh = plsc.VectorSubcoreMesh(core_axis_name="c", subcore_axis_name="s")
    half = jax.ShapeDtypeStruct((_NP, _H), _f32)

    @functools.partial(
        pl.kernel,
        out_type=(half, half),
        mesh=mesh,
        scratch_types=[
            pltpu.VMEM((_GW, _H), _f32),
            pltpu.VMEM((1, _GW), jnp.int32),
            pltpu.VMEM_SHARED((_NP, _H), _f32),
        ])
    def sk(lo_hbm, hi_hbm, r_hbm, z_hbm, olo_hbm, ohi_hbm, rows_v, idx_v, acc_sh):
        c = lax.axis_index("c")
        s = lax.axis_index("s")
        pltpu.sync_copy(z_hbm.at[pl.ds(s * _RS, _RS)],
                        acc_sh.at[pl.ds(s * _RS, _RS)])
        plsc.subcore_barrier()

        def scatter_from(src_hbm):
            @pl.loop(0, (_SG + _NSUB - 1) // _NSUB)
            def _(i):
                g = i * _NSUB + s

                @pl.when(g < _SG)
                def _():
                    pltpu.sync_copy(r_hbm.at[g], idx_v)
                    pltpu.sync_copy(src_hbm.at[pl.ds(g * _GW, _GW)], rows_v)
                    pltpu.sync_copy(rows_v, acc_sh.at[idx_v.at[0]], add=True)

        @pl.when(c == 0)
        def _():
            scatter_from(lo_hbm)

        @pl.when(c == 1)
        def _():
            scatter_from(hi_hbm)

        plsc.subcore_barrier()

        @pl.when(c == 0)
        def _():
            pltpu.sync_copy(acc_sh.at[pl.ds(s * _RS, _RS)],
                            olo_hbm.at[pl.ds(s * _RS, _RS)])

        @pl.when(c == 1)
        def _():
            pltpu.sync_copy(acc_sh.at[pl.ds(s * _RS, _RS)],
                            ohi_hbm.at[pl.ds(s * _RS, _RS)])

    return sk(ne_lo, ne_hi, recv2d, zeros)


def _node_body(n_ref, na_ref, alo_ref, ahi_ref, wlo_ref, whi_ref, w2_ref,
               b1_ref, b2_ref, g_ref, b_ref, out_ref):
    x = na_ref[...] + b1_ref[...]
    x = x + jnp.dot(alo_ref[...].astype(_bf16), wlo_ref[...],
                    preferred_element_type=_f32)
    x = x + jnp.dot(ahi_ref[...].astype(_bf16), whi_ref[...],
                    preferred_element_type=_f32)
    h = jnp.maximum(x, 0.0)
    y = jnp.dot(h.astype(_bf16), w2_ref[...], preferred_element_type=_f32)
    nn = _layer_norm(y + b2_ref[...], g_ref[...], b_ref[...])
    out_ref[...] = nn + n_ref[...]


def _node_mlp(nodes, na, alo, ahi, wlo, whi, w2n, b1, b2, g, b):
    full = lambda shp: pl.BlockSpec(shp, lambda i: (0, 0))
    return pl.pallas_call(
        _node_body,
        grid=(_N // _NB,),
        in_specs=[pl.BlockSpec((_NB, _D), lambda i: (i, 0)),
                  pl.BlockSpec((_NB, _D), lambda i: (i, 0)),
                  pl.BlockSpec((_NB, _H), lambda i: (i, 0)),
                  pl.BlockSpec((_NB, _H), lambda i: (i, 0)),
                  full((_H, _D)), full((_H, _D)), full((_D, _D)),
                  full((1, _D)), full((1, _D)), full((1, _D)), full((1, _D))],
        out_specs=pl.BlockSpec((_NB, _D), lambda i: (i, 0)),
        out_shape=jax.ShapeDtypeStruct((_N, _D), _f32),
    )(nodes, na, alo, ahi, wlo, whi, w2n, b1, b2, g, b)


def kernel(node_features, edge_features, senders, receivers,
           W1e, b1e, W2e, b2e, ge, be,
           W1n, b1n, W2n, b2n, gn, bn):
    ws = W1e[:_D].astype(_bf16)
    wr = W1e[_D:2 * _D].astype(_bf16)
    w1c = W1e[2 * _D:].astype(_bf16)
    w2e = W2e.astype(_bf16)
    wna = W1n[:_D].astype(_bf16)
    wlo = W1n[_D:_D + _H].astype(_bf16)
    whi = W1n[_D + _H:].astype(_bf16)
    w2n = W2n.astype(_bf16)
    row = lambda v: v.reshape(1, _D)

    ps, pr, na = _proj(node_features, ws, wr, wna)
    gs = _sc_gather(ps, senders.reshape(1, _E))
    gr = _sc_gather(pr, receivers.reshape(1, _E))
    out_edges, ne_lo, ne_hi = _edge_mlp(
        gs, gr, edge_features, w1c, w2e,
        row(b1e), row(b2e), row(ge), row(be))
    zeros = jnp.zeros((_NP, _H), _f32)
    agg_lo, agg_hi = _sc_scatter(ne_lo, ne_hi,
                                 receivers.reshape(_SG, 1, _GW), zeros)
    agg_lo, agg_hi = agg_lo[:_N], agg_hi[:_N]
    out_nodes = _node_mlp(
        node_features, na, agg_lo, agg_hi, wlo, whi, w2n,
        row(b1n), row(b2n), row(gn), row(bn))
    return (out_nodes, out_edges)
